# baseline (speedup 1.0000x reference)
import jax
import jax.numpy as jnp
from jax.experimental import pallas as pl
from jax.experimental.pallas import tpu as pltpu

S = 8
H = 256
E = S * S
C = 2
K = E // C


def _msgs_body(et_ref, gf_ref, wq_ref, wk_ref, *rest):
    w_refs = rest[:K]
    out_ref = rest[K]
    c = pl.program_id(0)

    gf = gf_ref[...]
    q = jnp.dot(gf, wq_ref[...], preferred_element_type=jnp.float32)
    k = jnp.dot(gf, wk_ref[...], preferred_element_type=jnp.float32)
    scores = jnp.dot(q, k.T, preferred_element_type=jnp.float32) * (1.0 / 16.0)
    scores = scores - jnp.max(scores, axis=-1, keepdims=True)
    ex = jnp.exp(scores)
    attn = ex / jnp.sum(ex, axis=-1, keepdims=True)
    wgf = (attn[:, :, None] * gf[:, None, :]).reshape(E, H)
    rows = jnp.where(c == 0, wgf[:K, :], wgf[K:, :])

    msgs = [jnp.dot(rows[j:j + 1, :], w_refs[j][0],
                    preferred_element_type=jnp.float32) for j in range(K)]
    total = jnp.concatenate(msgs[:S], axis=0)
    for b in range(1, K // S):
        total = total + jnp.concatenate(msgs[S * b:S * (b + 1)], axis=0)
    out_ref[0] = total


def _finish_body(p_ref, gf_ref, root_ref, rb_ref, lrel_ref, lroot_ref,
                 gb_ref, out_ref):
    gf = gf_ref[...]
    x1 = (p_ref[0] + p_ref[1]
          + jnp.dot(gf, root_ref[...], preferred_element_type=jnp.float32)
          + rb_ref[...])
    agg = jnp.broadcast_to(jnp.sum(x1, axis=0, keepdims=True), (S, H))
    out2 = (jnp.dot(agg, lrel_ref[...], preferred_element_type=jnp.float32)
            + jnp.dot(x1, lroot_ref[...], preferred_element_type=jnp.float32)
            + gb_ref[...])
    out_ref[:, :H] = out2
    out_ref[:, H:] = gf


def kernel(global_features, speaker, Wq, Wk, Wv, rgcn_weight, rgcn_root,
           rgcn_bias, gcn_lin_rel, gcn_lin_root, gcn_bias):
    del Wv
    spk = speaker.astype(jnp.int32)
    src = jnp.repeat(jnp.arange(S, dtype=jnp.int32), S)
    dst = jnp.tile(jnp.arange(S, dtype=jnp.int32), S)
    et = (spk[src] * S + spk[dst]) * 2 + (src >= dst).astype(jnp.int32)

    full = lambda shape: pl.BlockSpec(shape, lambda c, et_ref: (0,) * len(shape))
    w_specs = [pl.BlockSpec((1, H, H),
                            lambda c, et_ref, j=j: (et_ref[K * c + j], 0, 0))
               for j in range(K)]
    grid_spec = pltpu.PrefetchScalarGridSpec(
        num_scalar_prefetch=1,
        grid=(C,),
        in_specs=[
            full((S, H)),
            full((H, H)),
            full((H, H)),
            *w_specs,
        ],
        out_specs=pl.BlockSpec((1, S, H), lambda c, et_ref: (c, 0, 0)),
    )
    partials = pl.pallas_call(
        _msgs_body,
        grid_spec=grid_spec,
        out_shape=jax.ShapeDtypeStruct((C, S, H), jnp.float32),
        compiler_params=pltpu.CompilerParams(
            dimension_semantics=("parallel",)),
    )(et, global_features, Wq, Wk, *([rgcn_weight] * K))

    return pl.pallas_call(
        _finish_body,
        out_shape=jax.ShapeDtypeStruct((S, 2 * H), jnp.float32),
    )(partials, global_features, rgcn_root, rgcn_bias.reshape(1, H),
      gcn_lin_rel, gcn_lin_root, gcn_bias.reshape(1, H))


# speedup vs baseline: 1.0751x; 1.0751x over previous
import jax
import jax.numpy as jnp
from jax.experimental import pallas as pl
from jax.experimental.pallas import tpu as pltpu

S = 8
H = 256
E = S * S


def _body(et_ref, gf_ref, wq_ref, wk_ref, *rest):
    w_refs = rest[:E]
    root_ref, rb_ref, lrel_ref, lroot_ref, gb_ref, out_ref = rest[E:]

    gf = gf_ref[...]
    q = jnp.dot(gf, wq_ref[...], preferred_element_type=jnp.float32)
    k = jnp.dot(gf, wk_ref[...], preferred_element_type=jnp.float32)
    scores = jnp.dot(q, k.T, preferred_element_type=jnp.float32) * (1.0 / 16.0)
    scores = scores - jnp.max(scores, axis=-1, keepdims=True)
    ex = jnp.exp(scores)
    attn = ex / jnp.sum(ex, axis=-1, keepdims=True)
    wgf = (attn[:, :, None] * gf[:, None, :]).reshape(E, H)

    msgs = [jnp.dot(wgf[e:e + 1, :], w_refs[e][0],
                    preferred_element_type=jnp.float32) for e in range(E)]
    x1 = (jnp.dot(gf, root_ref[...], preferred_element_type=jnp.float32)
          + rb_ref[...])
    for b in range(S):
        x1 = x1 + jnp.concatenate(msgs[S * b:S * (b + 1)], axis=0)

    agg = jnp.broadcast_to(jnp.sum(x1, axis=0, keepdims=True), (S, H))
    out2 = (jnp.dot(agg, lrel_ref[...], preferred_element_type=jnp.float32)
            + jnp.dot(x1, lroot_ref[...], preferred_element_type=jnp.float32)
            + gb_ref[...])
    out_ref[:, :H] = out2
    out_ref[:, H:] = gf


def kernel(global_features, speaker, Wq, Wk, Wv, rgcn_weight, rgcn_root,
           rgcn_bias, gcn_lin_rel, gcn_lin_root, gcn_bias):
    del Wv
    spk = speaker.astype(jnp.int32)
    src = jnp.repeat(jnp.arange(S, dtype=jnp.int32), S)
    dst = jnp.tile(jnp.arange(S, dtype=jnp.int32), S)
    et = (spk[src] * S + spk[dst]) * 2 + (src >= dst).astype(jnp.int32)

    full = lambda shape: pl.BlockSpec(shape, lambda i, et_ref: (0,) * len(shape))
    w_specs = [pl.BlockSpec((1, H, H),
                            lambda i, et_ref, e=e: (et_ref[e], 0, 0))
               for e in range(E)]
    grid_spec = pltpu.PrefetchScalarGridSpec(
        num_scalar_prefetch=1,
        grid=(1,),
        in_specs=[
            full((S, H)),
            full((H, H)),
            full((H, H)),
            *w_specs,
            full((H, H)),
            full((1, H)),
            full((H, H)),
            full((H, H)),
            full((1, H)),
        ],
        out_specs=pl.BlockSpec((S, 2 * H), lambda i, et_ref: (0, 0)),
    )
    return pl.pallas_call(
        _body,
        grid_spec=grid_spec,
        out_shape=jax.ShapeDtypeStruct((S, 2 * H), jnp.float32),
    )(et, global_features, Wq, Wk, *([rgcn_weight] * E), rgcn_root,
      rgcn_bias.reshape(1, H), gcn_lin_rel, gcn_lin_root,
      gcn_bias.reshape(1, H))
